# Initial kernel scaffold; baseline (speedup 1.0000x reference)
import jax
import jax.numpy as jnp
from jax.experimental import pallas as pl


def kernel(x, pos, W_first, b_first, W_mid, b_mid, W_last, b_last):
    raise NotImplementedError("write your pallas kernel here")


# speedup vs baseline: 15.2330x; 15.2330x over previous
import functools

import jax
import jax.numpy as jnp
from jax import lax
from jax.experimental import pallas as pl
from jax.experimental.pallas import tpu as pltpu
from jax.experimental.pallas import tpu_sc as plsc

KNN = 16
ROWS_A = 128
ROWS_C = 512
GATHER_CHUNK = 2048


def _knn_precomp_kernel(posr_ref, posT_ref, sqr_ref, sqa_ref, x_ref,
                        p1t_ref, p2t_ref,
                        idx_ref, prec_ref, v_ref, *, n, rows):
    b = pl.program_id(0)
    pos_r = posr_ref[0]
    posT = posT_ref[0]
    dot = jax.lax.dot_general(pos_r, posT, (((1,), (0,)), ((), ())),
                              preferred_element_type=jnp.float32)
    d2 = (sqr_ref[0][:, 0][:, None] + sqa_ref[0]) - 2.0 * dot
    col = lax.broadcasted_iota(jnp.int32, (rows, n), 1)
    inf = jnp.float32(jnp.inf)
    ams = []
    for _ in range(KNN + 1):
        m = jnp.min(d2, axis=1)
        am = jnp.min(jnp.where(d2 == m[:, None], col, n), axis=1)
        ams.append(am)
        d2 = jnp.where(col == am[:, None], inf, d2)
    idx_ref[0] = jnp.stack(ams[1:], axis=1) + b * n
    x_r = x_ref[0]
    prec_ref[0] = jax.lax.dot_general(x_r, p1t_ref[...], (((1,), (0,)), ((), ())),
                                      preferred_element_type=jnp.float32)
    v_ref[0] = jax.lax.dot_general(x_r, p2t_ref[...], (((1,), (0,)), ((), ())),
                                   preferred_element_type=jnp.float32)


def _edge_mlp_kernel(prec_ref, vg_ref, x_ref, w2at_ref, w3at_ref, w3bt_ref,
                     b1_ref, b2_ref, b3_ref, out_ref, *, rows):
    prec = prec_ref[0]
    u = prec[:, 0:16]
    w2x = prec[:, 16:32]
    w3x = prec[:, 32:48]
    vg = vg_ref[0]
    uk = jnp.broadcast_to(u[:, None, :], (rows, KNN, 16)).reshape(rows * KNN, 16)
    w2k = jnp.broadcast_to(w2x[:, None, :], (rows, KNN, 16)).reshape(rows * KNN, 16)
    y1 = jax.nn.relu(uk + vg + b1_ref[0][None, :])
    y2 = jax.nn.relu(
        jax.lax.dot_general(y1, w2at_ref[...], (((1,), (0,)), ((), ())),
                            preferred_element_type=jnp.float32)
        + w2k + b2_ref[0][None, :])
    t3 = (jax.lax.dot_general(y2, w3at_ref[...], (((1,), (0,)), ((), ())),
                              preferred_element_type=jnp.float32)
          + jax.lax.dot_general(y1, w3bt_ref[...], (((1,), (0,)), ((), ())),
                                preferred_element_type=jnp.float32))
    m3 = jnp.max(t3.reshape(rows, KNN, 16), axis=1) + w3x + b3_ref[0][None, :]
    m2 = jnp.max(y2.reshape(rows, KNN, 16), axis=1)
    m1 = jnp.max(y1.reshape(rows, KNN, 16), axis=1)
    out_ref[0] = jnp.concatenate([m3, m2, m1, x_ref[0]], axis=1)


def _make_sc_gather(total, nw):
    per_w = total // nw
    nchunk = per_w // GATHER_CHUNK
    mesh = plsc.VectorSubcoreMesh(core_axis_name="c", subcore_axis_name="s")

    def body(table_hbm, idx_hbm, out_hbm, idx_v, rows_v, sem):
        nc = 2
        wid = lax.axis_index("s") * nc + lax.axis_index("c")
        for c in range(nchunk):
            base = wid * per_w + c * GATHER_CHUNK
            pltpu.sync_copy(idx_hbm.at[pl.ds(base, GATHER_CHUNK)], idx_v)
            pltpu.async_copy(table_hbm.at[idx_v], rows_v, sem).wait()
            pltpu.sync_copy(rows_v, out_hbm.at[pl.ds(base, GATHER_CHUNK)])

    return functools.partial(
        pl.kernel, body, mesh=mesh,
        compiler_params=pltpu.CompilerParams(use_tc_tiling_on_sc=False),
        out_type=jax.ShapeDtypeStruct((total, 16), jnp.float32),
        scratch_types=[
            pltpu.VMEM((GATHER_CHUNK,), jnp.int32),
            pltpu.VMEM((GATHER_CHUNK, 16), jnp.float32),
            pltpu.SemaphoreType.DMA,
        ])()


def kernel(x, pos, W_first, b_first, W_mid, b_mid, W_last, b_last):
    bsz, n, d = x.shape
    w1a, w1b, w1c = W_first[:, :d], W_first[:, d:2 * d], W_first[:, 2 * d:]
    u_w = w1a - w1c
    v_w = w1b + w1c
    w2a, w2b = W_mid[:, :16], W_mid[:, 16:]
    w3a, w3b, w3c = W_last[:, :16], W_last[:, 16:32], W_last[:, 32:]
    p1t = jnp.concatenate([u_w, w2b, w3c], axis=0).T
    p2t = v_w.T

    pos8 = jnp.pad(pos, ((0, 0), (0, 0), (0, 5)))
    pos8T = jnp.swapaxes(pos8, 1, 2)
    sq = jnp.sum(pos * pos, axis=-1)

    ra = ROWS_A
    idx, prec, v = pl.pallas_call(
        functools.partial(_knn_precomp_kernel, n=n, rows=ra),
        grid=(bsz, n // ra),
        in_specs=[
            pl.BlockSpec((1, ra, 8), lambda b, i: (b, i, 0)),
            pl.BlockSpec((1, 8, n), lambda b, i: (b, 0, 0)),
            pl.BlockSpec((1, ra, 1), lambda b, i: (b, i, 0)),
            pl.BlockSpec((1, 1, n), lambda b, i: (b, 0, 0)),
            pl.BlockSpec((1, ra, d), lambda b, i: (b, i, 0)),
            pl.BlockSpec((d, 48), lambda b, i: (0, 0)),
            pl.BlockSpec((d, 16), lambda b, i: (0, 0)),
        ],
        out_specs=[
            pl.BlockSpec((1, ra, KNN), lambda b, i: (b, i, 0)),
            pl.BlockSpec((1, ra, 48), lambda b, i: (b, i, 0)),
            pl.BlockSpec((1, ra, 16), lambda b, i: (b, i, 0)),
        ],
        out_shape=[
            jax.ShapeDtypeStruct((bsz, n, KNN), jnp.int32),
            jax.ShapeDtypeStruct((bsz, n, 48), jnp.float32),
            jax.ShapeDtypeStruct((bsz, n, 16), jnp.float32),
        ],
    )(pos8, pos8T, sq[:, :, None], sq[:, None, :], x, p1t, p2t)

    flat_idx = idx.reshape(bsz * n * KNN)
    table = v.reshape(bsz * n, 16)
    vg = _make_sc_gather(bsz * n * KNN, 32)(table, flat_idx)

    rc = ROWS_C
    out = pl.pallas_call(
        functools.partial(_edge_mlp_kernel, rows=rc),
        grid=(bsz, n // rc),
        in_specs=[
            pl.BlockSpec((1, rc, 48), lambda b, i: (b, i, 0)),
            pl.BlockSpec((1, rc * KNN, 16), lambda b, i: (b, i, 0)),
            pl.BlockSpec((1, rc, d), lambda b, i: (b, i, 0)),
            pl.BlockSpec((16, 16), lambda b, i: (0, 0)),
            pl.BlockSpec((16, 16), lambda b, i: (0, 0)),
            pl.BlockSpec((16, 16), lambda b, i: (0, 0)),
            pl.BlockSpec((1, 16), lambda b, i: (0, 0)),
            pl.BlockSpec((1, 16), lambda b, i: (0, 0)),
            pl.BlockSpec((1, 16), lambda b, i: (0, 0)),
        ],
        out_specs=pl.BlockSpec((1, rc, 48 + d), lambda b, i: (b, i, 0)),
        out_shape=jax.ShapeDtypeStruct((bsz, n, 48 + d), jnp.float32),
    )(prec, vg.reshape(bsz, n * KNN, 16), x,
      w2a.T, w3a.T, w3b.T,
      b_first[None, :], b_mid[None, :], b_last[None, :])
    return out


# speedup vs baseline: 18.1301x; 1.1902x over previous
import functools

import jax
import jax.numpy as jnp
from jax import lax
from jax.experimental import pallas as pl
from jax.experimental.pallas import tpu as pltpu
from jax.experimental.pallas import tpu_sc as plsc

KNN = 16
ROWS_A = 128
ROWS_C = 512
GATHER_CHUNK = 2048


def _knn_precomp_kernel(posr_ref, posT_ref, sqr_ref, sqa_ref, x_ref,
                        p1t_ref, p2t_ref,
                        idx_ref, prec_ref, v_ref, *, n, rows):
    b = pl.program_id(0)
    pos_r = posr_ref[0]
    posT = posT_ref[0]
    dot = jax.lax.dot_general(pos_r, posT, (((1,), (0,)), ((), ())),
                              preferred_element_type=jnp.float32)
    d2 = (sqr_ref[0][:, 0][:, None] + sqa_ref[0]) - 2.0 * dot
    colf = lax.broadcasted_iota(jnp.int32, (rows, n), 1).astype(jnp.float32)
    inf = jnp.float32(jnp.inf)
    nf = jnp.float32(n)
    ams = []
    amf = jnp.full((rows, 1), -1.0, jnp.float32)
    for _ in range(KNN + 1):
        d2 = jnp.where(colf == amf, inf, d2)
        m = jnp.min(d2, axis=1)
        amf = jnp.min(jnp.where(d2 == m[:, None], colf, nf), axis=1)[:, None]
        ams.append(amf)
    idx_f = jnp.concatenate(ams[1:], axis=1)
    idx_ref[0] = idx_f.astype(jnp.int32) + b * n
    x_r = x_ref[0]
    prec_ref[0] = jax.lax.dot_general(x_r, p1t_ref[...], (((1,), (0,)), ((), ())),
                                      preferred_element_type=jnp.float32)
    v_ref[0] = jax.lax.dot_general(x_r, p2t_ref[...], (((1,), (0,)), ((), ())),
                                   preferred_element_type=jnp.float32)


def _edge_mlp_kernel(prec_ref, vg_ref, x_ref, w2at_ref, w3at_ref, w3bt_ref,
                     b1_ref, b2_ref, b3_ref, out_ref, *, rows):
    prec = prec_ref[0]
    u = prec[:, 0:16]
    w2x = prec[:, 16:32]
    w3x = prec[:, 32:48]
    vg = vg_ref[0]
    uk = jnp.broadcast_to(u[:, None, :], (rows, KNN, 16)).reshape(rows * KNN, 16)
    w2k = jnp.broadcast_to(w2x[:, None, :], (rows, KNN, 16)).reshape(rows * KNN, 16)
    y1 = jax.nn.relu(uk + vg + b1_ref[0][None, :])
    y2 = jax.nn.relu(
        jax.lax.dot_general(y1, w2at_ref[...], (((1,), (0,)), ((), ())),
                            preferred_element_type=jnp.float32)
        + w2k + b2_ref[0][None, :])
    t3 = (jax.lax.dot_general(y2, w3at_ref[...], (((1,), (0,)), ((), ())),
                              preferred_element_type=jnp.float32)
          + jax.lax.dot_general(y1, w3bt_ref[...], (((1,), (0,)), ((), ())),
                                preferred_element_type=jnp.float32))
    m3 = jnp.max(t3.reshape(rows, KNN, 16), axis=1) + w3x + b3_ref[0][None, :]
    m2 = jnp.max(y2.reshape(rows, KNN, 16), axis=1)
    m1 = jnp.max(y1.reshape(rows, KNN, 16), axis=1)
    out_ref[0] = jnp.concatenate([m3, m2, m1, x_ref[0]], axis=1)


def _make_sc_gather(total, nw):
    per_w = total // nw
    nchunk = per_w // GATHER_CHUNK
    mesh = plsc.VectorSubcoreMesh(core_axis_name="c", subcore_axis_name="s")

    def body(table_hbm, idx_hbm, out_hbm, idx_v, rows_v, sem):
        nc = 2
        wid = lax.axis_index("s") * nc + lax.axis_index("c")
        for c in range(nchunk):
            base = wid * per_w + c * GATHER_CHUNK
            pltpu.sync_copy(idx_hbm.at[pl.ds(base, GATHER_CHUNK)], idx_v)
            pltpu.async_copy(table_hbm.at[idx_v], rows_v, sem).wait()
            pltpu.sync_copy(rows_v, out_hbm.at[pl.ds(base, GATHER_CHUNK)])

    return functools.partial(
        pl.kernel, body, mesh=mesh,
        compiler_params=pltpu.CompilerParams(use_tc_tiling_on_sc=False),
        out_type=jax.ShapeDtypeStruct((total, 16), jnp.float32),
        scratch_types=[
            pltpu.VMEM((GATHER_CHUNK,), jnp.int32),
            pltpu.VMEM((GATHER_CHUNK, 16), jnp.float32),
            pltpu.SemaphoreType.DMA,
        ])()


def kernel(x, pos, W_first, b_first, W_mid, b_mid, W_last, b_last):
    bsz, n, d = x.shape
    w1a, w1b, w1c = W_first[:, :d], W_first[:, d:2 * d], W_first[:, 2 * d:]
    u_w = w1a - w1c
    v_w = w1b + w1c
    w2a, w2b = W_mid[:, :16], W_mid[:, 16:]
    w3a, w3b, w3c = W_last[:, :16], W_last[:, 16:32], W_last[:, 32:]
    p1t = jnp.concatenate([u_w, w2b, w3c], axis=0).T
    p2t = v_w.T

    pos8 = jnp.pad(pos, ((0, 0), (0, 0), (0, 5)))
    pos8T = jnp.swapaxes(pos8, 1, 2)
    sq = jnp.sum(pos * pos, axis=-1)

    ra = ROWS_A
    idx, prec, v = pl.pallas_call(
        functools.partial(_knn_precomp_kernel, n=n, rows=ra),
        grid=(bsz, n // ra),
        in_specs=[
            pl.BlockSpec((1, ra, 8), lambda b, i: (b, i, 0)),
            pl.BlockSpec((1, 8, n), lambda b, i: (b, 0, 0)),
            pl.BlockSpec((1, ra, 1), lambda b, i: (b, i, 0)),
            pl.BlockSpec((1, 1, n), lambda b, i: (b, 0, 0)),
            pl.BlockSpec((1, ra, d), lambda b, i: (b, i, 0)),
            pl.BlockSpec((d, 48), lambda b, i: (0, 0)),
            pl.BlockSpec((d, 16), lambda b, i: (0, 0)),
        ],
        out_specs=[
            pl.BlockSpec((1, ra, KNN), lambda b, i: (b, i, 0)),
            pl.BlockSpec((1, ra, 48), lambda b, i: (b, i, 0)),
            pl.BlockSpec((1, ra, 16), lambda b, i: (b, i, 0)),
        ],
        out_shape=[
            jax.ShapeDtypeStruct((bsz, n, KNN), jnp.int32),
            jax.ShapeDtypeStruct((bsz, n, 48), jnp.float32),
            jax.ShapeDtypeStruct((bsz, n, 16), jnp.float32),
        ],
    )(pos8, pos8T, sq[:, :, None], sq[:, None, :], x, p1t, p2t)

    flat_idx = idx.reshape(bsz * n * KNN)
    table = v.reshape(bsz * n, 16)
    vg = _make_sc_gather(bsz * n * KNN, 32)(table, flat_idx)

    rc = ROWS_C
    out = pl.pallas_call(
        functools.partial(_edge_mlp_kernel, rows=rc),
        grid=(bsz, n // rc),
        in_specs=[
            pl.BlockSpec((1, rc, 48), lambda b, i: (b, i, 0)),
            pl.BlockSpec((1, rc * KNN, 16), lambda b, i: (b, i, 0)),
            pl.BlockSpec((1, rc, d), lambda b, i: (b, i, 0)),
            pl.BlockSpec((16, 16), lambda b, i: (0, 0)),
            pl.BlockSpec((16, 16), lambda b, i: (0, 0)),
            pl.BlockSpec((16, 16), lambda b, i: (0, 0)),
            pl.BlockSpec((1, 16), lambda b, i: (0, 0)),
            pl.BlockSpec((1, 16), lambda b, i: (0, 0)),
            pl.BlockSpec((1, 16), lambda b, i: (0, 0)),
        ],
        out_specs=pl.BlockSpec((1, rc, 48 + d), lambda b, i: (b, i, 0)),
        out_shape=jax.ShapeDtypeStruct((bsz, n, 48 + d), jnp.float32),
    )(prec, vg.reshape(bsz, n * KNN, 16), x,
      w2a.T, w3a.T, w3b.T,
      b_first[None, :], b_mid[None, :], b_last[None, :])
    return out


# speedup vs baseline: 28.8358x; 1.5905x over previous
import functools

import jax
import jax.numpy as jnp
from jax import lax
from jax.experimental import pallas as pl
from jax.experimental.pallas import tpu as pltpu
from jax.experimental.pallas import tpu_sc as plsc

KNN = 16
ROWS_A = 128
ROWS_C = 512
GATHER_CHUNK = 2048


LEVELS = 5
GROUP = 128


def _knn_precomp_kernel(posc_ref, posrT_ref, sqc_ref, sqr_ref, x_ref,
                        p1t_ref, p2t_ref,
                        idx_ref, prec_ref, v_ref, *, n, rows):
    b = pl.program_id(0)
    pos_c = posc_ref[0]
    pos_rT = posrT_ref[0]
    dot = jax.lax.dot_general(pos_c, pos_rT, (((1,), (0,)), ((), ())),
                              preferred_element_type=jnp.float32)
    d2T = (sqc_ref[0] + sqr_ref[0]) - 2.0 * dot
    inf = jnp.float32(jnp.inf)
    nf = jnp.float32(n)
    g = n // GROUP
    gfl = jnp.float32(g)
    wfl = jnp.float32(GROUP)

    V = d2T.reshape(g, GROUP, rows)
    wf = lax.broadcasted_iota(jnp.int32, (g, GROUP, rows), 1).astype(jnp.float32)
    Ms, As = [], []
    Vm = V
    for l in range(LEVELS):
        if l:
            Vm = jnp.where(wf == As[-1][:, None, :], inf, Vm)
        M = jnp.min(Vm, axis=1)
        A = jnp.min(jnp.where(Vm == M[:, None, :], wf, wfl), axis=1)
        Ms.append(M)
        As.append(A)

    gf = lax.broadcasted_iota(jnp.int32, (g, rows), 0).astype(jnp.float32)
    c = jnp.zeros((g, rows), jnp.float32)
    picks = []
    for _ in range(KNN + 1):
        cur = inf
        curA = wfl
        for l in reversed(range(LEVELS)):
            sel = c == jnp.float32(l)
            cur = jnp.where(sel, Ms[l], cur)
            curA = jnp.where(sel, As[l], curA)
        m = jnp.min(cur, axis=0)
        gsel = jnp.min(jnp.where(cur == m[None, :], gf, gfl), axis=0)
        wsel = jnp.min(jnp.where(gf == gsel[None, :], curA, wfl), axis=0)
        picks.append(gsel * jnp.float32(GROUP) + wsel)
        c = c + (gf == gsel[None, :]).astype(jnp.float32)
    idx_f = jnp.stack(picks[1:], axis=1)
    idx_ref[0] = idx_f.astype(jnp.int32) + b * n

    @pl.when(jnp.max(c) >= jnp.float32(LEVELS))
    def _fallback():
        colT = lax.broadcasted_iota(jnp.int32, (n, rows), 0).astype(jnp.float32)
        d2w = d2T
        amf = jnp.full((1, rows), -1.0, jnp.float32)
        pk = []
        for _ in range(KNN + 1):
            d2w = jnp.where(colT == amf, inf, d2w)
            m = jnp.min(d2w, axis=0)
            am = jnp.min(jnp.where(d2w == m[None, :], colT, nf), axis=0)
            amf = am[None, :]
            pk.append(am)
        idx2 = jnp.stack(pk[1:], axis=1)
        idx_ref[0] = idx2.astype(jnp.int32) + b * n
    x_r = x_ref[0]
    prec_ref[0] = jax.lax.dot_general(x_r, p1t_ref[...], (((1,), (0,)), ((), ())),
                                      preferred_element_type=jnp.float32)
    v_ref[0] = jax.lax.dot_general(x_r, p2t_ref[...], (((1,), (0,)), ((), ())),
                                   preferred_element_type=jnp.float32)


def _edge_mlp_kernel(prec_ref, vg_ref, x_ref, w2at_ref, w3at_ref, w3bt_ref,
                     b1_ref, b2_ref, b3_ref, out_ref, *, rows):
    prec = prec_ref[0]
    u = prec[:, 0:16]
    w2x = prec[:, 16:32]
    w3x = prec[:, 32:48]
    vg = vg_ref[0]
    uk = jnp.broadcast_to(u[:, None, :], (rows, KNN, 16)).reshape(rows * KNN, 16)
    w2k = jnp.broadcast_to(w2x[:, None, :], (rows, KNN, 16)).reshape(rows * KNN, 16)
    y1 = jax.nn.relu(uk + vg + b1_ref[0][None, :])
    y2 = jax.nn.relu(
        jax.lax.dot_general(y1, w2at_ref[...], (((1,), (0,)), ((), ())),
                            preferred_element_type=jnp.float32)
        + w2k + b2_ref[0][None, :])
    t3 = (jax.lax.dot_general(y2, w3at_ref[...], (((1,), (0,)), ((), ())),
                              preferred_element_type=jnp.float32)
          + jax.lax.dot_general(y1, w3bt_ref[...], (((1,), (0,)), ((), ())),
                                preferred_element_type=jnp.float32))
    m3 = jnp.max(t3.reshape(rows, KNN, 16), axis=1) + w3x + b3_ref[0][None, :]
    m2 = jnp.max(y2.reshape(rows, KNN, 16), axis=1)
    m1 = jnp.max(y1.reshape(rows, KNN, 16), axis=1)
    out_ref[0] = jnp.concatenate([m3, m2, m1, x_ref[0]], axis=1)


def _make_sc_gather(total, nw):
    per_w = total // nw
    nchunk = per_w // GATHER_CHUNK
    mesh = plsc.VectorSubcoreMesh(core_axis_name="c", subcore_axis_name="s")

    def body(table_hbm, idx_hbm, out_hbm, idx_v, rows_v, sem):
        nc = 2
        wid = lax.axis_index("s") * nc + lax.axis_index("c")
        for c in range(nchunk):
            base = wid * per_w + c * GATHER_CHUNK
            pltpu.sync_copy(idx_hbm.at[pl.ds(base, GATHER_CHUNK)], idx_v)
            pltpu.async_copy(table_hbm.at[idx_v], rows_v, sem).wait()
            pltpu.sync_copy(rows_v, out_hbm.at[pl.ds(base, GATHER_CHUNK)])

    return functools.partial(
        pl.kernel, body, mesh=mesh,
        compiler_params=pltpu.CompilerParams(use_tc_tiling_on_sc=False),
        out_type=jax.ShapeDtypeStruct((total, 16), jnp.float32),
        scratch_types=[
            pltpu.VMEM((GATHER_CHUNK,), jnp.int32),
            pltpu.VMEM((GATHER_CHUNK, 16), jnp.float32),
            pltpu.SemaphoreType.DMA,
        ])()


def kernel(x, pos, W_first, b_first, W_mid, b_mid, W_last, b_last):
    bsz, n, d = x.shape
    w1a, w1b, w1c = W_first[:, :d], W_first[:, d:2 * d], W_first[:, 2 * d:]
    u_w = w1a - w1c
    v_w = w1b + w1c
    w2a, w2b = W_mid[:, :16], W_mid[:, 16:]
    w3a, w3b, w3c = W_last[:, :16], W_last[:, 16:32], W_last[:, 32:]
    p1t = jnp.concatenate([u_w, w2b, w3c], axis=0).T
    p2t = v_w.T

    pos8 = jnp.pad(pos, ((0, 0), (0, 0), (0, 5)))
    pos8T = jnp.swapaxes(pos8, 1, 2)
    sq = jnp.sum(pos * pos, axis=-1)

    ra = ROWS_A
    idx, prec, v = pl.pallas_call(
        functools.partial(_knn_precomp_kernel, n=n, rows=ra),
        grid=(bsz, n // ra),
        in_specs=[
            pl.BlockSpec((1, n, 8), lambda b, i: (b, 0, 0)),
            pl.BlockSpec((1, 8, ra), lambda b, i: (b, 0, i)),
            pl.BlockSpec((1, n, 1), lambda b, i: (b, 0, 0)),
            pl.BlockSpec((1, 1, ra), lambda b, i: (b, 0, i)),
            pl.BlockSpec((1, ra, d), lambda b, i: (b, i, 0)),
            pl.BlockSpec((d, 48), lambda b, i: (0, 0)),
            pl.BlockSpec((d, 16), lambda b, i: (0, 0)),
        ],
        out_specs=[
            pl.BlockSpec((1, ra, KNN), lambda b, i: (b, i, 0)),
            pl.BlockSpec((1, ra, 48), lambda b, i: (b, i, 0)),
            pl.BlockSpec((1, ra, 16), lambda b, i: (b, i, 0)),
        ],
        out_shape=[
            jax.ShapeDtypeStruct((bsz, n, KNN), jnp.int32),
            jax.ShapeDtypeStruct((bsz, n, 48), jnp.float32),
            jax.ShapeDtypeStruct((bsz, n, 16), jnp.float32),
        ],
    )(pos8, pos8T, sq[:, :, None], sq[:, None, :], x, p1t, p2t)

    flat_idx = idx.reshape(bsz * n * KNN)
    table = v.reshape(bsz * n, 16)
    vg = _make_sc_gather(bsz * n * KNN, 32)(table, flat_idx)

    rc = ROWS_C
    out = pl.pallas_call(
        functools.partial(_edge_mlp_kernel, rows=rc),
        grid=(bsz, n // rc),
        in_specs=[
            pl.BlockSpec((1, rc, 48), lambda b, i: (b, i, 0)),
            pl.BlockSpec((1, rc * KNN, 16), lambda b, i: (b, i, 0)),
            pl.BlockSpec((1, rc, d), lambda b, i: (b, i, 0)),
            pl.BlockSpec((16, 16), lambda b, i: (0, 0)),
            pl.BlockSpec((16, 16), lambda b, i: (0, 0)),
            pl.BlockSpec((16, 16), lambda b, i: (0, 0)),
            pl.BlockSpec((1, 16), lambda b, i: (0, 0)),
            pl.BlockSpec((1, 16), lambda b, i: (0, 0)),
            pl.BlockSpec((1, 16), lambda b, i: (0, 0)),
        ],
        out_specs=pl.BlockSpec((1, rc, 48 + d), lambda b, i: (b, i, 0)),
        out_shape=jax.ShapeDtypeStruct((bsz, n, 48 + d), jnp.float32),
    )(prec, vg.reshape(bsz, n * KNN, 16), x,
      w2a.T, w3a.T, w3b.T,
      b_first[None, :], b_mid[None, :], b_last[None, :])
    return out


# speedup vs baseline: 31.3017x; 1.0855x over previous
import functools

import jax
import jax.numpy as jnp
from jax import lax
from jax.experimental import pallas as pl
from jax.experimental.pallas import tpu as pltpu
from jax.experimental.pallas import tpu_sc as plsc

KNN = 16
ROWS_A = 128
ROWS_C = 512
GATHER_CHUNK = 2048


LEVELS = 6
GROUP = 128


def _knn_precomp_kernel(posc_ref, posrT_ref, sqc_ref, sqr_ref, x_ref,
                        p1t_ref, p2t_ref,
                        idx_ref, prec_ref, v_ref, *, n, rows):
    b = pl.program_id(0)
    pos_c = posc_ref[0]
    pos_rT = posrT_ref[0]
    dot = jax.lax.dot_general(pos_c, pos_rT, (((1,), (0,)), ((), ())),
                              preferred_element_type=jnp.float32)
    d2T = (sqc_ref[0] + sqr_ref[0]) - 2.0 * dot
    inf = jnp.float32(jnp.inf)
    nf = jnp.float32(n)
    g = n // GROUP
    gfl = jnp.float32(g)
    wfl = jnp.float32(GROUP)

    V = d2T.reshape(g, GROUP, rows)
    wf = lax.broadcasted_iota(jnp.int32, (g, GROUP, rows), 1).astype(jnp.float32)
    Ms, As = [], []
    Vm = V
    for l in range(LEVELS):
        if l:
            Vm = jnp.where(wf == As[-1][:, None, :], inf, Vm)
        M = jnp.min(Vm, axis=1)
        A = jnp.min(jnp.where(Vm == M[:, None, :], wf, wfl), axis=1)
        Ms.append(M)
        As.append(A)

    gf = lax.broadcasted_iota(jnp.int32, (g, rows), 0).astype(jnp.float32)
    c = jnp.zeros((g, rows), jnp.float32)
    picks = []
    for _ in range(KNN + 1):
        cur = inf
        curA = wfl
        for l in reversed(range(LEVELS)):
            sel = c == jnp.float32(l)
            cur = jnp.where(sel, Ms[l], cur)
            curA = jnp.where(sel, As[l], curA)
        m = jnp.min(cur, axis=0)
        gsel = jnp.min(jnp.where(cur == m[None, :], gf, gfl), axis=0)
        wsel = jnp.min(jnp.where(gf == gsel[None, :], curA, wfl), axis=0)
        picks.append(gsel * jnp.float32(GROUP) + wsel)
        c = c + (gf == gsel[None, :]).astype(jnp.float32)
    idx_f = jnp.stack(picks[1:], axis=1)
    idx_ref[0] = idx_f.astype(jnp.int32) + b * n

    @pl.when(jnp.max(c) >= jnp.float32(LEVELS))
    def _fallback():
        colT = lax.broadcasted_iota(jnp.int32, (n, rows), 0).astype(jnp.float32)
        d2w = d2T
        amf = jnp.full((1, rows), -1.0, jnp.float32)
        pk = []
        for _ in range(KNN + 1):
            d2w = jnp.where(colT == amf, inf, d2w)
            m = jnp.min(d2w, axis=0)
            am = jnp.min(jnp.where(d2w == m[None, :], colT, nf), axis=0)
            amf = am[None, :]
            pk.append(am)
        idx2 = jnp.stack(pk[1:], axis=1)
        idx_ref[0] = idx2.astype(jnp.int32) + b * n
    x_r = x_ref[0]
    prec_ref[0] = jax.lax.dot_general(x_r, p1t_ref[...], (((1,), (0,)), ((), ())),
                                      preferred_element_type=jnp.float32)
    v_ref[0] = jax.lax.dot_general(x_r, p2t_ref[...], (((1,), (0,)), ((), ())),
                                   preferred_element_type=jnp.float32)


def _edge_mlp_kernel(prec_ref, vg_ref, x_ref, w2at_ref, w3at_ref, w3bt_ref,
                     b1_ref, b2_ref, b3_ref, out_ref, *, rows):
    prec = prec_ref[0]
    u = prec[:, 0:16]
    w2x = prec[:, 16:32]
    w3x = prec[:, 32:48]
    vg = vg_ref[0]
    uk = jnp.broadcast_to(u[:, None, :], (rows, KNN, 16)).reshape(rows * KNN, 16)
    w2k = jnp.broadcast_to(w2x[:, None, :], (rows, KNN, 16)).reshape(rows * KNN, 16)
    y1 = jax.nn.relu(uk + vg + b1_ref[0][None, :])
    y2 = jax.nn.relu(
        jax.lax.dot_general(y1, w2at_ref[...], (((1,), (0,)), ((), ())),
                            preferred_element_type=jnp.float32)
        + w2k + b2_ref[0][None, :])
    t3 = (jax.lax.dot_general(y2, w3at_ref[...], (((1,), (0,)), ((), ())),
                              preferred_element_type=jnp.float32)
          + jax.lax.dot_general(y1, w3bt_ref[...], (((1,), (0,)), ((), ())),
                                preferred_element_type=jnp.float32))
    def _maxk(y2d):
        t = y2d.reshape(rows, KNN, 16)
        r = KNN // 2
        while r >= 1:
            t = jnp.maximum(t[:, :r, :], t[:, r:2 * r, :])
            r //= 2
        return t[:, 0, :]

    m3 = _maxk(t3) + w3x + b3_ref[0][None, :]
    m2 = _maxk(y2)
    m1 = _maxk(y1)
    out_ref[0] = jnp.concatenate([m3, m2, m1, x_ref[0]], axis=1)


def _make_sc_gather(total, nw):
    per_w = total // nw
    nchunk = per_w // GATHER_CHUNK
    mesh = plsc.VectorSubcoreMesh(core_axis_name="c", subcore_axis_name="s")

    def body(table_hbm, idx_hbm, out_hbm, idx_v, rows_v, sem):
        nc = 2
        wid = lax.axis_index("s") * nc + lax.axis_index("c")
        for c in range(nchunk):
            base = wid * per_w + c * GATHER_CHUNK
            pltpu.sync_copy(idx_hbm.at[pl.ds(base, GATHER_CHUNK)], idx_v)
            pltpu.async_copy(table_hbm.at[idx_v], rows_v, sem).wait()
            pltpu.sync_copy(rows_v, out_hbm.at[pl.ds(base, GATHER_CHUNK)])

    return functools.partial(
        pl.kernel, body, mesh=mesh,
        compiler_params=pltpu.CompilerParams(use_tc_tiling_on_sc=False),
        out_type=jax.ShapeDtypeStruct((total, 16), jnp.float32),
        scratch_types=[
            pltpu.VMEM((GATHER_CHUNK,), jnp.int32),
            pltpu.VMEM((GATHER_CHUNK, 16), jnp.float32),
            pltpu.SemaphoreType.DMA,
        ])()


def kernel(x, pos, W_first, b_first, W_mid, b_mid, W_last, b_last):
    bsz, n, d = x.shape
    w1a, w1b, w1c = W_first[:, :d], W_first[:, d:2 * d], W_first[:, 2 * d:]
    u_w = w1a - w1c
    v_w = w1b + w1c
    w2a, w2b = W_mid[:, :16], W_mid[:, 16:]
    w3a, w3b, w3c = W_last[:, :16], W_last[:, 16:32], W_last[:, 32:]
    p1t = jnp.concatenate([u_w, w2b, w3c], axis=0).T
    p2t = v_w.T

    pos8 = jnp.pad(pos, ((0, 0), (0, 0), (0, 5)))
    pos8T = jnp.swapaxes(pos8, 1, 2)
    sq = jnp.sum(pos * pos, axis=-1)

    ra = ROWS_A
    idx, prec, v = pl.pallas_call(
        functools.partial(_knn_precomp_kernel, n=n, rows=ra),
        grid=(bsz, n // ra),
        in_specs=[
            pl.BlockSpec((1, n, 8), lambda b, i: (b, 0, 0)),
            pl.BlockSpec((1, 8, ra), lambda b, i: (b, 0, i)),
            pl.BlockSpec((1, n, 1), lambda b, i: (b, 0, 0)),
            pl.BlockSpec((1, 1, ra), lambda b, i: (b, 0, i)),
            pl.BlockSpec((1, ra, d), lambda b, i: (b, i, 0)),
            pl.BlockSpec((d, 48), lambda b, i: (0, 0)),
            pl.BlockSpec((d, 16), lambda b, i: (0, 0)),
        ],
        out_specs=[
            pl.BlockSpec((1, ra, KNN), lambda b, i: (b, i, 0)),
            pl.BlockSpec((1, ra, 48), lambda b, i: (b, i, 0)),
            pl.BlockSpec((1, ra, 16), lambda b, i: (b, i, 0)),
        ],
        out_shape=[
            jax.ShapeDtypeStruct((bsz, n, KNN), jnp.int32),
            jax.ShapeDtypeStruct((bsz, n, 48), jnp.float32),
            jax.ShapeDtypeStruct((bsz, n, 16), jnp.float32),
        ],
    )(pos8, pos8T, sq[:, :, None], sq[:, None, :], x, p1t, p2t)

    flat_idx = idx.reshape(bsz * n * KNN)
    table = v.reshape(bsz * n, 16)
    vg = _make_sc_gather(bsz * n * KNN, 32)(table, flat_idx)

    rc = ROWS_C
    out = pl.pallas_call(
        functools.partial(_edge_mlp_kernel, rows=rc),
        grid=(bsz, n // rc),
        in_specs=[
            pl.BlockSpec((1, rc, 48), lambda b, i: (b, i, 0)),
            pl.BlockSpec((1, rc * KNN, 16), lambda b, i: (b, i, 0)),
            pl.BlockSpec((1, rc, d), lambda b, i: (b, i, 0)),
            pl.BlockSpec((16, 16), lambda b, i: (0, 0)),
            pl.BlockSpec((16, 16), lambda b, i: (0, 0)),
            pl.BlockSpec((16, 16), lambda b, i: (0, 0)),
            pl.BlockSpec((1, 16), lambda b, i: (0, 0)),
            pl.BlockSpec((1, 16), lambda b, i: (0, 0)),
            pl.BlockSpec((1, 16), lambda b, i: (0, 0)),
        ],
        out_specs=pl.BlockSpec((1, rc, 48 + d), lambda b, i: (b, i, 0)),
        out_shape=jax.ShapeDtypeStruct((bsz, n, 48 + d), jnp.float32),
    )(prec, vg.reshape(bsz, n * KNN, 16), x,
      w2a.T, w3a.T, w3b.T,
      b_first[None, :], b_mid[None, :], b_last[None, :])
    return out


# speedup vs baseline: 34.3218x; 1.0965x over previous
import functools

import jax
import jax.numpy as jnp
from jax import lax
from jax.experimental import pallas as pl
from jax.experimental.pallas import tpu as pltpu
from jax.experimental.pallas import tpu_sc as plsc

KNN = 16
ROWS_A = 128
ROWS_C = 512
GATHER_CHUNK = 2048


LEVELS = 6
GROUP = 128


def _knn_precomp_kernel(posc_ref, posrT_ref, sqc_ref, sqr_ref, x_ref,
                        p1t_ref, p2t_ref,
                        idx_ref, prec_ref, v_ref, *, n, rows):
    b = pl.program_id(0)
    pos_c = posc_ref[0]
    pos_rT = posrT_ref[0]
    dot = jax.lax.dot_general(pos_c, pos_rT, (((1,), (0,)), ((), ())),
                              preferred_element_type=jnp.float32)
    d2T = (sqc_ref[0] + sqr_ref[0]) - 2.0 * dot
    inf = jnp.float32(jnp.inf)
    nf = jnp.float32(n)
    g = n // GROUP
    gfl = jnp.float32(g)
    wfl = jnp.float32(GROUP)

    V = d2T.reshape(g, GROUP, rows)
    wf = lax.broadcasted_iota(jnp.int32, (g, GROUP, rows), 1).astype(jnp.float32)
    Ms, As = [], []
    Vm = V
    for l in range(LEVELS):
        if l:
            Vm = jnp.where(wf == As[-1][:, None, :], inf, Vm)
        M = jnp.min(Vm, axis=1)
        A = jnp.min(jnp.where(Vm == M[:, None, :], wf, wfl), axis=1)
        Ms.append(M)
        As.append(A)

    gf = lax.broadcasted_iota(jnp.int32, (g, rows), 0).astype(jnp.float32)
    c = jnp.zeros((g, rows), jnp.float32)
    picks = []
    for _ in range(KNN + 1):
        cur = inf
        curA = wfl
        for l in reversed(range(LEVELS)):
            sel = c == jnp.float32(l)
            cur = jnp.where(sel, Ms[l], cur)
            curA = jnp.where(sel, As[l], curA)
        m = jnp.min(cur, axis=0)
        gsel = jnp.min(jnp.where(cur == m[None, :], gf, gfl), axis=0)
        wsel = jnp.min(jnp.where(gf == gsel[None, :], curA, wfl), axis=0)
        picks.append(gsel * jnp.float32(GROUP) + wsel)
        c = c + (gf == gsel[None, :]).astype(jnp.float32)
    idx_f = jnp.stack(picks[1:], axis=1)
    idx_ref[0] = idx_f.astype(jnp.int32) + b * n

    @pl.when(jnp.max(c) >= jnp.float32(LEVELS))
    def _fallback():
        colT = lax.broadcasted_iota(jnp.int32, (n, rows), 0).astype(jnp.float32)
        d2w = d2T
        amf = jnp.full((1, rows), -1.0, jnp.float32)
        pk = []
        for _ in range(KNN + 1):
            d2w = jnp.where(colT == amf, inf, d2w)
            m = jnp.min(d2w, axis=0)
            am = jnp.min(jnp.where(d2w == m[None, :], colT, nf), axis=0)
            amf = am[None, :]
            pk.append(am)
        idx2 = jnp.stack(pk[1:], axis=1)
        idx_ref[0] = idx2.astype(jnp.int32) + b * n
    x_r = x_ref[0]
    prec_ref[0] = jax.lax.dot_general(x_r, p1t_ref[...], (((1,), (0,)), ((), ())),
                                      preferred_element_type=jnp.float32)
    v_ref[0] = jax.lax.dot_general(x_r, p2t_ref[...], (((1,), (0,)), ((), ())),
                                   preferred_element_type=jnp.float32)


def _edge_mlp_kernel(prec_ref, vg_ref, x_ref, w2at_ref, w3at_ref, w3bt_ref,
                     b1_ref, b2_ref, b3_ref, out_ref, *, rows):
    prec = prec_ref[0]
    u = prec[:, 0:16]
    w2x = prec[:, 16:32]
    w3x = prec[:, 32:48]
    vg = vg_ref[0]
    uk = jnp.concatenate([u] * KNN, axis=1)
    w2k = jnp.concatenate([w2x] * KNN, axis=1)
    y1 = jax.nn.relu(uk + vg + b1_ref[0][None, :])
    y2 = jax.nn.relu(
        jax.lax.dot_general(y1, w2at_ref[...], (((1,), (0,)), ((), ())),
                            preferred_element_type=jnp.float32)
        + w2k + b2_ref[0][None, :])
    t3 = (jax.lax.dot_general(y2, w3at_ref[...], (((1,), (0,)), ((), ())),
                              preferred_element_type=jnp.float32)
          + jax.lax.dot_general(y1, w3bt_ref[...], (((1,), (0,)), ((), ())),
                                preferred_element_type=jnp.float32))
    def _maxk(t):
        w = (KNN * 16) // 2
        while w >= 16:
            t = jnp.maximum(t[:, :w], t[:, w:2 * w])
            w //= 2
        return t

    m3 = _maxk(t3) + w3x + b3_ref[0][None, :]
    m2 = _maxk(y2)
    m1 = _maxk(y1)
    out_ref[0] = jnp.concatenate([m3, m2, m1, x_ref[0]], axis=1)


def _make_sc_gather(total, nw):
    per_w = total // nw
    nchunk = per_w // GATHER_CHUNK
    mesh = plsc.VectorSubcoreMesh(core_axis_name="c", subcore_axis_name="s")

    def body(table_hbm, idx_hbm, out_hbm, idx_v, rows_v, sem):
        nc = 2
        wid = lax.axis_index("s") * nc + lax.axis_index("c")
        for c in range(nchunk):
            base = wid * per_w + c * GATHER_CHUNK
            pltpu.sync_copy(idx_hbm.at[pl.ds(base, GATHER_CHUNK)], idx_v)
            pltpu.async_copy(table_hbm.at[idx_v], rows_v, sem).wait()
            pltpu.sync_copy(rows_v, out_hbm.at[pl.ds(base, GATHER_CHUNK)])

    return functools.partial(
        pl.kernel, body, mesh=mesh,
        compiler_params=pltpu.CompilerParams(use_tc_tiling_on_sc=False),
        out_type=jax.ShapeDtypeStruct((total, 16), jnp.float32),
        scratch_types=[
            pltpu.VMEM((GATHER_CHUNK,), jnp.int32),
            pltpu.VMEM((GATHER_CHUNK, 16), jnp.float32),
            pltpu.SemaphoreType.DMA,
        ])()


def kernel(x, pos, W_first, b_first, W_mid, b_mid, W_last, b_last):
    bsz, n, d = x.shape
    w1a, w1b, w1c = W_first[:, :d], W_first[:, d:2 * d], W_first[:, 2 * d:]
    u_w = w1a - w1c
    v_w = w1b + w1c
    w2a, w2b = W_mid[:, :16], W_mid[:, 16:]
    w3a, w3b, w3c = W_last[:, :16], W_last[:, 16:32], W_last[:, 32:]
    p1t = jnp.concatenate([u_w, w2b, w3c], axis=0).T
    p2t = v_w.T

    pos8 = jnp.pad(pos, ((0, 0), (0, 0), (0, 5)))
    pos8T = jnp.swapaxes(pos8, 1, 2)
    sq = jnp.sum(pos * pos, axis=-1)

    ra = ROWS_A
    idx, prec, v = pl.pallas_call(
        functools.partial(_knn_precomp_kernel, n=n, rows=ra),
        grid=(bsz, n // ra),
        in_specs=[
            pl.BlockSpec((1, n, 8), lambda b, i: (b, 0, 0)),
            pl.BlockSpec((1, 8, ra), lambda b, i: (b, 0, i)),
            pl.BlockSpec((1, n, 1), lambda b, i: (b, 0, 0)),
            pl.BlockSpec((1, 1, ra), lambda b, i: (b, 0, i)),
            pl.BlockSpec((1, ra, d), lambda b, i: (b, i, 0)),
            pl.BlockSpec((d, 48), lambda b, i: (0, 0)),
            pl.BlockSpec((d, 16), lambda b, i: (0, 0)),
        ],
        out_specs=[
            pl.BlockSpec((1, ra, KNN), lambda b, i: (b, i, 0)),
            pl.BlockSpec((1, ra, 48), lambda b, i: (b, i, 0)),
            pl.BlockSpec((1, ra, 16), lambda b, i: (b, i, 0)),
        ],
        out_shape=[
            jax.ShapeDtypeStruct((bsz, n, KNN), jnp.int32),
            jax.ShapeDtypeStruct((bsz, n, 48), jnp.float32),
            jax.ShapeDtypeStruct((bsz, n, 16), jnp.float32),
        ],
    )(pos8, pos8T, sq[:, :, None], sq[:, None, :], x, p1t, p2t)

    flat_idx = idx.reshape(bsz * n * KNN)
    table = v.reshape(bsz * n, 16)
    vg = _make_sc_gather(bsz * n * KNN, 32)(table, flat_idx)

    rc = ROWS_C
    out = pl.pallas_call(
        functools.partial(_edge_mlp_kernel, rows=rc),
        grid=(bsz, n // rc),
        in_specs=[
            pl.BlockSpec((1, rc, 48), lambda b, i: (b, i, 0)),
            pl.BlockSpec((1, rc, KNN * 16), lambda b, i: (b, i, 0)),
            pl.BlockSpec((1, rc, d), lambda b, i: (b, i, 0)),
            pl.BlockSpec((KNN * 16, KNN * 16), lambda b, i: (0, 0)),
            pl.BlockSpec((KNN * 16, KNN * 16), lambda b, i: (0, 0)),
            pl.BlockSpec((KNN * 16, KNN * 16), lambda b, i: (0, 0)),
            pl.BlockSpec((1, KNN * 16), lambda b, i: (0, 0)),
            pl.BlockSpec((1, KNN * 16), lambda b, i: (0, 0)),
            pl.BlockSpec((1, 16), lambda b, i: (0, 0)),
        ],
        out_specs=pl.BlockSpec((1, rc, 48 + d), lambda b, i: (b, i, 0)),
        out_shape=jax.ShapeDtypeStruct((bsz, n, 48 + d), jnp.float32),
    )(prec, vg.reshape(bsz, n, KNN * 16), x,
      jnp.kron(jnp.eye(KNN, dtype=jnp.float32), w2a.T),
      jnp.kron(jnp.eye(KNN, dtype=jnp.float32), w3a.T),
      jnp.kron(jnp.eye(KNN, dtype=jnp.float32), w3b.T),
      jnp.tile(b_first, KNN)[None, :], jnp.tile(b_mid, KNN)[None, :],
      b_last[None, :])
    return out
